# baseline (speedup 1.0000x reference)
import math

import jax
import jax.numpy as jnp
from jax import lax
from jax.experimental import pallas as pl
from jax.experimental.pallas import tpu as pltpu

CONTRAST = 3.38589
SQRT2 = math.sqrt(2.0)
B, C, H_IN, W_IN = 2, 3, 384, 384
H_OUT, W_OUT = 384, 384
P = H_OUT * W_OUT
WR = 3
H_PAD = H_IN + 2 * WR + 1
W_PAD = W_IN + 2 * WR + 1
K = H_PAD * W_PAD

_SIG1 = math.tanh(0.5 * CONTRAST * 0.5)
_A3 = -3.0
_A2 = (45739.0 + 7164.0 * SQRT2) / 10319.0
_A0 = (-8926.0 - 14328.0 * SQRT2) / 10319.0
_MIR = (-103.0 - 36.0 * SQRT2) / (7.0 + 72.0 * SQRT2)

PT = 1024


def _inverse_sigmoid(q):
    sig0 = -_SIG1
    slope = (1.0 / _SIG1 + sig0) * 0.25 * CONTRAST
    oos = 1.0 / slope
    res_low = q * oos
    res_high = q * oos + (1.0 - oos)
    ssq = jnp.clip(2.0 * _SIG1 * q + sig0, -0.999999, 0.999999)
    res_mid = (2.0 / CONTRAST) * (0.5 * jnp.log((1.0 + ssq) / (1.0 - ssq))) + 0.5
    return jnp.where(q <= 0.0, res_low, jnp.where(q >= 1.0, res_high, res_mid))


def _extended_sigmoid(q):
    slope = (1.0 / _SIG1 - _SIG1) * 0.25 * CONTRAST
    res_low = slope * q
    res_high = slope * q + (1.0 - slope)
    res_mid = 0.5 / _SIG1 * jnp.tanh(0.5 * CONTRAST * q - 0.25 * CONTRAST) + 0.5
    return jnp.where(q <= 0.0, res_low, jnp.where(q >= 1.0, res_high, res_mid))


def _mitchell(x):
    ax = jnp.abs(x)
    ax2 = ax * ax
    ax3 = ax2 * ax
    v1 = (7.0 / 6.0) * ax3 - 2.0 * ax2 + 8.0 / 9.0
    v2 = (-7.0 / 18.0) * ax3 + 2.0 * ax2 - (10.0 / 3.0) * ax + 16.0 / 9.0
    m1 = (ax < 1.0).astype(x.dtype)
    m2 = ((ax >= 1.0) & (ax < 2.0)).astype(x.dtype)
    return v1 * m1 + v2 * m2


def _pack_body(x_ref, o_ref):
    x = x_ref[...]
    s = _inverse_sigmoid(x)
    ru = lax.bitcast_convert_type(x.astype(jnp.bfloat16), jnp.uint16)
    su = lax.bitcast_convert_type(s.astype(jnp.bfloat16), jnp.uint16)
    o_ref[...] = (ru.astype(jnp.uint32) << 16) | su.astype(jnp.uint32)


def _pack_planes(img_pad):
    return pl.pallas_call(
        _pack_body,
        grid=(B, C),
        in_specs=[pl.BlockSpec((1, 1, H_PAD, W_PAD), lambda b, c: (b, c, 0, 0))],
        out_specs=pl.BlockSpec((1, 1, H_PAD, W_PAD), lambda b, c: (b, c, 0, 0)),
        out_shape=jax.ShapeDtypeStruct((B, C, H_PAD, W_PAD), jnp.uint32),
    )(img_pad)


def _math_body(g_ref, e_ref, o_ref):
    f32 = jnp.float32
    lane = lax.broadcasted_iota(jnp.int32, (1, 108), 1)
    oxl = ((lane % 6) - 2).astype(f32)
    oyl = ((lane // 18) - 2).astype(f32)
    chl = (lane // 6) % 3

    g = g_ref[0]
    raw = lax.bitcast_convert_type(g & jnp.uint32(0xFFFF0000), f32)
    sig = lax.bitcast_convert_type(g << 16, f32)

    e = e_ref[0]
    j00 = e[:, 0:1]
    j01 = e[:, 1:2]
    j10 = e[:, 2:3]
    j11 = e[:, 3:4]
    fx = e[:, 4:5]
    fy = e[:, 5:6]

    rel_x = fx - oxl
    rel_y = fy - oyl
    wm = _mitchell(rel_x) * _mitchell(rel_y)

    det = j00 * j11 - j01 * j10 + 1e-8
    a = j11 / det
    b = -j01 / det
    c = -j10 / det
    d = j00 / det
    n11 = a * a + b * b
    n12 = a * c + b * d
    n22 = c * c + d * d
    frob = n11 + n22
    disc = frob * frob - 4.0 / (det * det)
    sq = jnp.sqrt(jnp.maximum(disc, 0.0))
    s1 = 0.5 * (frob + sq)
    major = jnp.sqrt(jnp.maximum(s1, 1.0))
    minor = jnp.sqrt(jnp.maximum(0.5 * (frob - sq), 1.0))
    d1 = s1 - n11
    d2 = s1 - n22
    cond = d1 * d1 >= d2 * d2
    t11 = jnp.where(cond, n12, d2)
    t21 = jnp.where(cond, d1, n12)
    norm = jnp.sqrt(t11 * t11 + t21 * t21)
    pos = norm > 0.0
    sn = jnp.where(pos, norm, 1.0)
    u11 = jnp.where(pos, t11 / sn, 1.0)
    u21 = jnp.where(pos, t21 / sn, 0.0)
    cmx = u11 / major
    cmy = u21 / major
    cnx = -u21 / minor
    cny = u11 / minor
    theta = 1.0 / (major * minor)
    need = (frob + sq) > 2.0

    q1 = rel_x * cmx + rel_y * cmy
    q2 = rel_x * cnx + rel_y * cny
    r2 = q1 * q1 + q2 * q2
    r = jnp.sqrt(r2 + 1e-8)
    w_in = r2 * (_A3 * r + _A2) + _A0
    w_out = (r + _MIR) * (r - 2.0) * (r - 2.0)
    we = w_in * (r2 < 1.0).astype(f32) + w_out * ((r2 >= 1.0) & (r2 < 4.0)).astype(f32)

    wsum = jnp.sum(we, axis=1, keepdims=True) * (1.0 / 3.0) + 1e-8
    pm = sig * wm
    pe = raw * we
    outs = []
    for cc in range(3):
        mk = (chl == cc).astype(f32)
        msum = jnp.sum(pm * mk, axis=1, keepdims=True)
        esum = jnp.sum(pe * mk, axis=1, keepdims=True)
        mv = _extended_sigmoid(msum)
        ev = esum / wsum
        bl = theta * mv + (1.0 - theta) * ev
        outs.append(jnp.where(need, bl, mv))
    o_ref[0] = jnp.concatenate(outs, axis=1)


def _math(g, e):
    return pl.pallas_call(
        _math_body,
        grid=(B, P // PT),
        in_specs=[
            pl.BlockSpec((1, PT, 108), lambda b, i: (b, i, 0)),
            pl.BlockSpec((1, PT, 6), lambda b, i: (b, i, 0)),
        ],
        out_specs=pl.BlockSpec((1, PT, 3), lambda b, i: (b, i, 0)),
        out_shape=jax.ShapeDtypeStruct((B, P, 3), jnp.float32),
    )(g, e)


def kernel(image, grid):
    f32 = jnp.float32
    img_pad = jnp.pad(image, ((0, 0), (0, 0), (WR, WR + 1), (WR, WR + 1)),
                      mode="edge")
    vpk = _pack_planes(img_pad)

    flat = vpk.reshape(B, C, K)
    flatp = jnp.pad(flat, ((0, 0), (0, 0), (0, 8)))
    tab = jnp.stack([flatp[:, :, j:j + K] for j in range(6)], axis=-1)
    tab = tab.transpose(0, 2, 1, 3).reshape(B, K, 18)

    gx = grid[..., 0].reshape(B, P)
    gy = grid[..., 1].reshape(B, P)
    ix = jnp.clip(jnp.floor(gx).astype(jnp.int32), 0, W_IN - 2)
    iy = jnp.clip(jnp.floor(gy).astype(jnp.int32), 0, H_IN - 2)
    fx = gx - (ix.astype(f32) + 0.5)
    fy = gy - (iy.astype(f32) + 0.5)
    base = (iy + 1) * W_PAD + (ix + 1)

    gpx = jnp.pad(grid, ((0, 0), (0, 0), (1, 1), (0, 0)), mode="edge")
    dx = (gpx[:, :, 2:, :] - gpx[:, :, :-2, :]) * 0.5
    gpy = jnp.pad(grid, ((0, 0), (1, 1), (0, 0), (0, 0)), mode="edge")
    dy = (gpy[:, 2:, :, :] - gpy[:, :-2, :, :]) * 0.5
    e = jnp.stack([
        dx[..., 0].reshape(B, P), dy[..., 0].reshape(B, P),
        dx[..., 1].reshape(B, P), dy[..., 1].reshape(B, P),
        fx, fy,
    ], axis=-1)

    idx6 = base[..., None] + jnp.arange(6, dtype=jnp.int32)[None, None, :] * W_PAD
    idx6 = idx6.reshape(B, P * 6, 1)
    g = jnp.take_along_axis(tab, jnp.broadcast_to(idx6, (B, P * 6, 18)), axis=1)
    g = g.reshape(B, P, 108)

    out = _math(g, e)
    return out.transpose(0, 2, 1).reshape(B, C, H_OUT, W_OUT)


# speedup vs baseline: 203.2666x; 203.2666x over previous
import math

import functools

import jax
import jax.numpy as jnp
from jax import lax
from jax.experimental import pallas as pl
from jax.experimental.pallas import tpu as pltpu
from jax.experimental.pallas import tpu_sc as plsc

CONTRAST = 3.38589
SQRT2 = math.sqrt(2.0)
B, C, H_IN, W_IN = 2, 3, 384, 384
H_OUT, W_OUT = 384, 384
P = H_OUT * W_OUT
WR = 3
H_PAD = H_IN + 2 * WR + 1
W_PAD = W_IN + 2 * WR + 1
K = H_PAD * W_PAD

_SIG1 = math.tanh(0.5 * CONTRAST * 0.5)
_A3 = -3.0
_A2 = (45739.0 + 7164.0 * SQRT2) / 10319.0
_A0 = (-8926.0 - 14328.0 * SQRT2) / 10319.0
_MIR = (-103.0 - 36.0 * SQRT2) / (7.0 + 72.0 * SQRT2)

PT = 1024


def _inverse_sigmoid(q):
    sig0 = -_SIG1
    slope = (1.0 / _SIG1 + sig0) * 0.25 * CONTRAST
    oos = 1.0 / slope
    res_low = q * oos
    res_high = q * oos + (1.0 - oos)
    ssq = jnp.clip(2.0 * _SIG1 * q + sig0, -0.999999, 0.999999)
    res_mid = (2.0 / CONTRAST) * (0.5 * jnp.log((1.0 + ssq) / (1.0 - ssq))) + 0.5
    return jnp.where(q <= 0.0, res_low, jnp.where(q >= 1.0, res_high, res_mid))


def _extended_sigmoid(q):
    slope = (1.0 / _SIG1 - _SIG1) * 0.25 * CONTRAST
    res_low = slope * q
    res_high = slope * q + (1.0 - slope)
    res_mid = 0.5 / _SIG1 * jnp.tanh(0.5 * CONTRAST * q - 0.25 * CONTRAST) + 0.5
    return jnp.where(q <= 0.0, res_low, jnp.where(q >= 1.0, res_high, res_mid))


def _mitchell(x):
    ax = jnp.abs(x)
    ax2 = ax * ax
    ax3 = ax2 * ax
    v1 = (7.0 / 6.0) * ax3 - 2.0 * ax2 + 8.0 / 9.0
    v2 = (-7.0 / 18.0) * ax3 + 2.0 * ax2 - (10.0 / 3.0) * ax + 16.0 / 9.0
    m1 = (ax < 1.0).astype(x.dtype)
    m2 = ((ax >= 1.0) & (ax < 2.0)).astype(x.dtype)
    return v1 * m1 + v2 * m2


def _pack_body(x_ref, o_ref):
    x = x_ref[...]
    s = _inverse_sigmoid(x)
    ru = lax.bitcast_convert_type(x.astype(jnp.bfloat16), jnp.uint16)
    su = lax.bitcast_convert_type(s.astype(jnp.bfloat16), jnp.uint16)
    o_ref[...] = (ru.astype(jnp.uint32) << 16) | su.astype(jnp.uint32)


def _pack_planes(img_pad):
    return pl.pallas_call(
        _pack_body,
        grid=(B, C),
        in_specs=[pl.BlockSpec((1, 1, H_PAD, W_PAD), lambda b, c: (b, c, 0, 0))],
        out_specs=pl.BlockSpec((1, 1, H_PAD, W_PAD), lambda b, c: (b, c, 0, 0)),
        out_shape=jax.ShapeDtypeStruct((B, C, H_PAD, W_PAD), jnp.uint32),
    )(img_pad)


NRG = B * P
NW = 32
GPW = NRG // NW
IDXR_PW = GPW // 128
SUBG = 4
CH = SUBG * 128
NCHUNK = GPW // CH


def _gather_body(tab_hbm, idx_hbm, out_hbm, idx_v, rows_v, sem):
    wid = lax.axis_index("s") * 2 + lax.axis_index("c")
    pltpu.sync_copy(idx_hbm.at[pl.ds(wid * IDXR_PW, IDXR_PW)], idx_v)

    def chunk(ci, carry):
        cops = [
            pltpu.async_copy(tab_hbm.at[idx_v.at[ci * SUBG + i]],
                             rows_v.at[pl.ds(i * 128, 128)], sem)
            for i in range(SUBG)
        ]
        for cp in cops:
            cp.wait()
        pltpu.sync_copy(
            rows_v, out_hbm.at[pl.ds(wid * GPW + ci * CH, CH)])
        return carry

    lax.fori_loop(0, NCHUNK, chunk, 0)


@functools.lru_cache(maxsize=None)
def _make_sc_gather():
    return pl.kernel(
        _gather_body,
        out_type=jax.ShapeDtypeStruct((NRG, 128), jnp.uint32),
        mesh=plsc.VectorSubcoreMesh(core_axis_name="c", subcore_axis_name="s"),
        scratch_types=[
            pltpu.VMEM((IDXR_PW, 128), jnp.int32),
            pltpu.VMEM((CH, 128), jnp.uint32),
            pltpu.SemaphoreType.DMA,
        ],
    )


def _sc_gather(tab, idx):
    return _make_sc_gather()(tab, idx)


def _math_body(g_ref, e_ref, o_ref):
    f32 = jnp.float32
    lane = lax.broadcasted_iota(jnp.int32, (1, 128), 1)
    xsl = (lane % 21) // 3
    oxl = (xsl - 2).astype(f32)
    oyl = ((lane // 21) - 2).astype(f32)
    chl = lane % 3
    valid = (lane < 126) & (xsl < 6)

    g = g_ref[0]
    raw = lax.bitcast_convert_type(g & jnp.uint32(0xFFFF0000), f32)
    sig = lax.bitcast_convert_type(g << 16, f32)

    e = e_ref[0]
    j00 = e[:, 0:1]
    j01 = e[:, 1:2]
    j10 = e[:, 2:3]
    j11 = e[:, 3:4]
    fx = e[:, 4:5]
    fy = e[:, 5:6]

    rel_x = fx - oxl
    rel_y = fy - oyl
    wm = _mitchell(rel_x) * _mitchell(rel_y)

    det = j00 * j11 - j01 * j10 + 1e-8
    a = j11 / det
    b = -j01 / det
    c = -j10 / det
    d = j00 / det
    n11 = a * a + b * b
    n12 = a * c + b * d
    n22 = c * c + d * d
    frob = n11 + n22
    disc = frob * frob - 4.0 / (det * det)
    sq = jnp.sqrt(jnp.maximum(disc, 0.0))
    s1 = 0.5 * (frob + sq)
    major = jnp.sqrt(jnp.maximum(s1, 1.0))
    minor = jnp.sqrt(jnp.maximum(0.5 * (frob - sq), 1.0))
    d1 = s1 - n11
    d2 = s1 - n22
    cond = d1 * d1 >= d2 * d2
    t11 = jnp.where(cond, n12, d2)
    t21 = jnp.where(cond, d1, n12)
    norm = jnp.sqrt(t11 * t11 + t21 * t21)
    pos = norm > 0.0
    sn = jnp.where(pos, norm, 1.0)
    u11 = jnp.where(pos, t11 / sn, 1.0)
    u21 = jnp.where(pos, t21 / sn, 0.0)
    cmx = u11 / major
    cmy = u21 / major
    cnx = -u21 / minor
    cny = u11 / minor
    theta = 1.0 / (major * minor)
    need = (frob + sq) > 2.0

    q1 = rel_x * cmx + rel_y * cmy
    q2 = rel_x * cnx + rel_y * cny
    r2 = q1 * q1 + q2 * q2
    r = jnp.sqrt(r2 + 1e-8)
    w_in = r2 * (_A3 * r + _A2) + _A0
    w_out = (r + _MIR) * (r - 2.0) * (r - 2.0)
    we = w_in * (r2 < 1.0).astype(f32) + w_out * ((r2 >= 1.0) & (r2 < 4.0)).astype(f32)

    wsum = jnp.sum(we * ((chl == 0) & valid).astype(f32),
                   axis=1, keepdims=True) + 1e-8
    pm = sig * wm
    pe = raw * we
    outs = []
    for cc in range(3):
        mk = ((chl == cc) & valid).astype(f32)
        msum = jnp.sum(pm * mk, axis=1, keepdims=True)
        esum = jnp.sum(pe * mk, axis=1, keepdims=True)
        mv = _extended_sigmoid(msum)
        ev = esum / wsum
        bl = theta * mv + (1.0 - theta) * ev
        outs.append(jnp.where(need, bl, mv))
    o_ref[0] = jnp.concatenate(outs, axis=1)


def _math(g, e):
    return pl.pallas_call(
        _math_body,
        grid=(B, P // PT),
        in_specs=[
            pl.BlockSpec((1, PT, 128), lambda b, i: (b, i, 0)),
            pl.BlockSpec((1, PT, 6), lambda b, i: (b, i, 0)),
        ],
        out_specs=pl.BlockSpec((1, PT, 3), lambda b, i: (b, i, 0)),
        out_shape=jax.ShapeDtypeStruct((B, P, 3), jnp.float32),
    )(g, e)


def kernel(image, grid):
    f32 = jnp.float32
    img_pad = jnp.pad(image, ((0, 0), (0, 0), (WR, WR + 1), (WR, WR + 1)),
                      mode="edge")
    vpk = _pack_planes(img_pad)

    flat = vpk.reshape(B, C, K)
    flatp = jnp.pad(flat, ((0, 0), (0, 0), (0, 6 * W_PAD + 7)))
    tab = jnp.stack([flatp[:, :, dy * W_PAD + xs: dy * W_PAD + xs + K]
                     for dy in range(6) for xs in range(7)], axis=1)
    tab = tab.transpose(0, 3, 1, 2).reshape(B, K, 126)
    tab = jnp.pad(tab, ((0, 0), (0, 0), (0, 2)))

    gx = grid[..., 0].reshape(B, P)
    gy = grid[..., 1].reshape(B, P)
    ix = jnp.clip(jnp.floor(gx).astype(jnp.int32), 0, W_IN - 2)
    iy = jnp.clip(jnp.floor(gy).astype(jnp.int32), 0, H_IN - 2)
    fx = gx - (ix.astype(f32) + 0.5)
    fy = gy - (iy.astype(f32) + 0.5)
    base = (iy + 1) * W_PAD + (ix + 1)

    gpx = jnp.pad(grid, ((0, 0), (0, 0), (1, 1), (0, 0)), mode="edge")
    dx = (gpx[:, :, 2:, :] - gpx[:, :, :-2, :]) * 0.5
    gpy = jnp.pad(grid, ((0, 0), (1, 1), (0, 0), (0, 0)), mode="edge")
    dy = (gpy[:, 2:, :, :] - gpy[:, :-2, :, :]) * 0.5
    e = jnp.stack([
        dx[..., 0].reshape(B, P), dy[..., 0].reshape(B, P),
        dx[..., 1].reshape(B, P), dy[..., 1].reshape(B, P),
        fx, fy,
    ], axis=-1)

    idx = base + (jnp.arange(B, dtype=jnp.int32) * K)[:, None]
    g = _sc_gather(tab.reshape(B * K, 128), idx.reshape(NRG // 128, 128))
    g = g.reshape(B, P, 128)

    out = _math(g, e)
    return out.transpose(0, 2, 1).reshape(B, C, H_OUT, W_OUT)


# speedup vs baseline: 322.7004x; 1.5876x over previous
import math

import functools

import jax
import jax.numpy as jnp
from jax import lax
from jax.experimental import pallas as pl
from jax.experimental.pallas import tpu as pltpu
from jax.experimental.pallas import tpu_sc as plsc

CONTRAST = 3.38589
SQRT2 = math.sqrt(2.0)
B, C, H_IN, W_IN = 2, 3, 384, 384
H_OUT, W_OUT = 384, 384
P = H_OUT * W_OUT
WR = 3
H_PAD = H_IN + 2 * WR + 1
W_PAD = W_IN + 2 * WR + 1
K = H_PAD * W_PAD

_SIG1 = math.tanh(0.5 * CONTRAST * 0.5)
_A3 = -3.0
_A2 = (45739.0 + 7164.0 * SQRT2) / 10319.0
_A0 = (-8926.0 - 14328.0 * SQRT2) / 10319.0
_MIR = (-103.0 - 36.0 * SQRT2) / (7.0 + 72.0 * SQRT2)

PT = 1024


def _inverse_sigmoid(q):
    sig0 = -_SIG1
    slope = (1.0 / _SIG1 + sig0) * 0.25 * CONTRAST
    oos = 1.0 / slope
    res_low = q * oos
    res_high = q * oos + (1.0 - oos)
    ssq = jnp.clip(2.0 * _SIG1 * q + sig0, -0.999999, 0.999999)
    res_mid = (2.0 / CONTRAST) * (0.5 * jnp.log((1.0 + ssq) / (1.0 - ssq))) + 0.5
    return jnp.where(q <= 0.0, res_low, jnp.where(q >= 1.0, res_high, res_mid))


def _extended_sigmoid(q):
    slope = (1.0 / _SIG1 - _SIG1) * 0.25 * CONTRAST
    res_low = slope * q
    res_high = slope * q + (1.0 - slope)
    res_mid = 0.5 / _SIG1 * jnp.tanh(0.5 * CONTRAST * q - 0.25 * CONTRAST) + 0.5
    return jnp.where(q <= 0.0, res_low, jnp.where(q >= 1.0, res_high, res_mid))


def _mitchell(x):
    ax = jnp.abs(x)
    ax2 = ax * ax
    ax3 = ax2 * ax
    v1 = (7.0 / 6.0) * ax3 - 2.0 * ax2 + 8.0 / 9.0
    v2 = (-7.0 / 18.0) * ax3 + 2.0 * ax2 - (10.0 / 3.0) * ax + 16.0 / 9.0
    m1 = (ax < 1.0).astype(x.dtype)
    m2 = ((ax >= 1.0) & (ax < 2.0)).astype(x.dtype)
    return v1 * m1 + v2 * m2


def _pack_body(x_ref, o_ref):
    x = x_ref[...]
    s = _inverse_sigmoid(x)
    ru = lax.bitcast_convert_type(x.astype(jnp.bfloat16), jnp.uint16)
    su = lax.bitcast_convert_type(s.astype(jnp.bfloat16), jnp.uint16)
    o_ref[...] = (ru.astype(jnp.uint32) << 16) | su.astype(jnp.uint32)


def _pack_planes(img_pad):
    return pl.pallas_call(
        _pack_body,
        grid=(B, C),
        in_specs=[pl.BlockSpec((1, 1, H_PAD, W_PAD), lambda b, c: (b, c, 0, 0))],
        out_specs=pl.BlockSpec((1, 1, H_PAD, W_PAD), lambda b, c: (b, c, 0, 0)),
        out_shape=jax.ShapeDtypeStruct((B, C, H_PAD, W_PAD), jnp.uint32),
    )(img_pad)


NRG = B * P
NW = 32
GPW = NRG // NW
IDXR_PW = GPW // 128
SUBG = 4
CH = SUBG * 128
NCHUNK = GPW // CH


def _gather_body(tab_hbm, idx_hbm, out_hbm, idx_v, rows_v, sem):
    wid = lax.axis_index("s") * 2 + lax.axis_index("c")
    pltpu.sync_copy(idx_hbm.at[pl.ds(wid * IDXR_PW, IDXR_PW)], idx_v)

    def chunk(ci, carry):
        cops = [
            pltpu.async_copy(tab_hbm.at[idx_v.at[ci * SUBG + i]],
                             rows_v.at[pl.ds(i * 128, 128)], sem)
            for i in range(SUBG)
        ]
        for cp in cops:
            cp.wait()
        pltpu.sync_copy(
            rows_v, out_hbm.at[pl.ds(wid * GPW + ci * CH, CH)])
        return carry

    lax.fori_loop(0, NCHUNK, chunk, 0)


@functools.lru_cache(maxsize=None)
def _make_sc_gather():
    return pl.kernel(
        _gather_body,
        out_type=jax.ShapeDtypeStruct((NRG, 128), jnp.uint32),
        mesh=plsc.VectorSubcoreMesh(core_axis_name="c", subcore_axis_name="s"),
        scratch_types=[
            pltpu.VMEM((IDXR_PW, 128), jnp.int32),
            pltpu.VMEM((CH, 128), jnp.uint32),
            pltpu.SemaphoreType.DMA,
        ],
    )


def _sc_gather(tab, idx):
    return _make_sc_gather()(tab, idx)


PTA = 2048


def _wprep_body(e_ref, o_ref):
    f32 = jnp.float32
    e = e_ref[0]
    j00 = e[0:1]
    j01 = e[1:2]
    j10 = e[2:3]
    j11 = e[3:4]
    fx = e[4:5]
    fy = e[5:6]

    det = j00 * j11 - j01 * j10 + 1e-8
    a = j11 / det
    b = -j01 / det
    c = -j10 / det
    d = j00 / det
    n11 = a * a + b * b
    n12 = a * c + b * d
    n22 = c * c + d * d
    frob = n11 + n22
    disc = frob * frob - 4.0 / (det * det)
    sq = jnp.sqrt(jnp.maximum(disc, 0.0))
    s1 = 0.5 * (frob + sq)
    major = jnp.sqrt(jnp.maximum(s1, 1.0))
    minor = jnp.sqrt(jnp.maximum(0.5 * (frob - sq), 1.0))
    d1 = s1 - n11
    d2 = s1 - n22
    cond = d1 * d1 >= d2 * d2
    t11 = jnp.where(cond, n12, d2)
    t21 = jnp.where(cond, d1, n12)
    norm = jnp.sqrt(t11 * t11 + t21 * t21)
    pos = norm > 0.0
    sn = jnp.where(pos, norm, 1.0)
    u11 = jnp.where(pos, t11 / sn, 1.0)
    u21 = jnp.where(pos, t21 / sn, 0.0)
    cmx = u11 / major
    cmy = u21 / major
    cnx = -u21 / minor
    cny = u11 / minor
    theta = 1.0 / (major * minor)
    need = ((frob + sq) > 2.0).astype(f32)

    mx = [_mitchell(fx - (xs - 2.0)) for xs in range(7)]
    my = [_mitchell(fy - (dy - 2.0)) for dy in range(6)]
    zero = jnp.zeros_like(fx)
    o_ref[0] = jnp.concatenate(
        mx + my + [cmx, cmy, cnx, cny, fx, fy, theta, need,
                   zero, zero, zero], axis=0)


def _wprep(e2):
    return pl.pallas_call(
        _wprep_body,
        grid=(B, P // PTA),
        in_specs=[pl.BlockSpec((1, 6, PTA), lambda b, i: (b, 0, i))],
        out_specs=pl.BlockSpec((1, 24, PTA), lambda b, i: (b, 0, i)),
        out_shape=jax.ShapeDtypeStruct((B, 24, P), jnp.float32),
    )(e2)


def _taps_body(g_ref, w_ref, o_ref):
    f32 = jnp.float32
    lane = lax.broadcasted_iota(jnp.int32, (1, 128), 1)
    xsl = (lane % 21) // 3
    oxl = (xsl - 2).astype(f32)
    oyl = ((lane // 21) - 2).astype(f32)

    g = g_ref[0]
    raw = lax.bitcast_convert_type(g & jnp.uint32(0xFFFF0000), f32)
    sig = lax.bitcast_convert_type(g << 16, f32)

    wt = jnp.transpose(w_ref[0])
    xtake = xsl
    ytake = jnp.minimum(lane // 21, 5) + 7
    mx_e = jnp.take_along_axis(wt, jnp.broadcast_to(xtake, wt.shape[:1] + (128,)), axis=1)
    my_e = jnp.take_along_axis(wt, jnp.broadcast_to(ytake, wt.shape[:1] + (128,)), axis=1)
    wm = mx_e * my_e
    cmx = wt[:, 13:14]
    cmy = wt[:, 14:15]
    cnx = wt[:, 15:16]
    cny = wt[:, 16:17]
    fx = wt[:, 17:18]
    fy = wt[:, 18:19]

    rel_x = fx - oxl
    rel_y = fy - oyl
    q1 = rel_x * cmx + rel_y * cmy
    q2 = rel_x * cnx + rel_y * cny
    r2 = q1 * q1 + q2 * q2
    r = jnp.sqrt(r2 + 1e-8)
    w_in = r2 * (_A3 * r + _A2) + _A0
    w_out = (r + _MIR) * (r - 2.0) * (r - 2.0)
    we = (w_in * (r2 < 1.0).astype(f32)
          + w_out * ((r2 >= 1.0) & (r2 < 4.0)).astype(f32))

    pm = sig * wm
    pe = raw * we

    ri = lax.broadcasted_iota(jnp.int32, (128, 8), 0)
    ci = lax.broadcasted_iota(jnp.int32, (128, 8), 1)
    ch_r = ri % 3
    xs_r = (ri % 21) // 3
    val_r = (ri < 126) & (xs_r < 6)
    m1 = ((ci < 3) & (ch_r == ci) & val_r).astype(f32)
    m2 = ((ci >= 3) & (ci < 6) & (ch_r == ci - 3) & val_r).astype(f32)
    m3 = ((ci == 6) & (ch_r == 0) & val_r).astype(f32)
    out = (jnp.dot(pm, m1, preferred_element_type=f32)
           + jnp.dot(pe, m2, preferred_element_type=f32)
           + jnp.dot(we, m3, preferred_element_type=f32))
    o_ref[0] = out


def _taps(g, w):
    return pl.pallas_call(
        _taps_body,
        grid=(B, P // PT),
        in_specs=[
            pl.BlockSpec((1, PT, 128), lambda b, i: (b, i, 0)),
            pl.BlockSpec((1, 24, PT), lambda b, i: (b, 0, i)),
        ],
        out_specs=pl.BlockSpec((1, PT, 8), lambda b, i: (b, i, 0)),
        out_shape=jax.ShapeDtypeStruct((B, P, 8), jnp.float32),
    )(g, w)


def _final_body(w_ref, s_ref, o_ref):
    theta = w_ref[0, 3:4]
    need = w_ref[0, 4:5]
    msum = s_ref[0, 0:3]
    esum = s_ref[0, 3:6]
    wsum = s_ref[0, 6:7] + 1e-8
    mv = _extended_sigmoid(msum)
    ev = esum / wsum
    bl = theta * mv + (1.0 - theta) * ev
    o_ref[0] = jnp.where(need > 0.5, bl, mv)


def _final(w, st):
    return pl.pallas_call(
        _final_body,
        grid=(B, P // PTA),
        in_specs=[
            pl.BlockSpec((1, 8, PTA), lambda b, i: (b, 2, i)),
            pl.BlockSpec((1, 8, PTA), lambda b, i: (b, 0, i)),
        ],
        out_specs=pl.BlockSpec((1, 3, PTA), lambda b, i: (b, 0, i)),
        out_shape=jax.ShapeDtypeStruct((B, 3, P), jnp.float32),
    )(w, st)


def kernel(image, grid):
    f32 = jnp.float32
    img_pad = jnp.pad(image, ((0, 0), (0, 0), (WR, WR + 1), (WR, WR + 1)),
                      mode="edge")
    vpk = _pack_planes(img_pad)

    flat = vpk.reshape(B, C, K)
    flatp = jnp.pad(flat, ((0, 0), (0, 0), (0, 6 * W_PAD + 7)))
    tab = jnp.stack([flatp[:, :, dy * W_PAD + xs: dy * W_PAD + xs + K]
                     for dy in range(6) for xs in range(7)], axis=1)
    tab = tab.transpose(0, 3, 1, 2).reshape(B, K, 126)
    tab = jnp.pad(tab, ((0, 0), (0, 0), (0, 2)))

    gx = grid[..., 0].reshape(B, P)
    gy = grid[..., 1].reshape(B, P)
    ix = jnp.clip(jnp.floor(gx).astype(jnp.int32), 0, W_IN - 2)
    iy = jnp.clip(jnp.floor(gy).astype(jnp.int32), 0, H_IN - 2)
    fx = gx - (ix.astype(f32) + 0.5)
    fy = gy - (iy.astype(f32) + 0.5)
    base = (iy + 1) * W_PAD + (ix + 1)

    gpx = jnp.pad(grid, ((0, 0), (0, 0), (1, 1), (0, 0)), mode="edge")
    dx = (gpx[:, :, 2:, :] - gpx[:, :, :-2, :]) * 0.5
    gpy = jnp.pad(grid, ((0, 0), (1, 1), (0, 0), (0, 0)), mode="edge")
    dy = (gpy[:, 2:, :, :] - gpy[:, :-2, :, :]) * 0.5
    e2 = jnp.stack([
        dx[..., 0].reshape(B, P), dy[..., 0].reshape(B, P),
        dx[..., 1].reshape(B, P), dy[..., 1].reshape(B, P),
        fx, fy,
    ], axis=1)

    idx = base + (jnp.arange(B, dtype=jnp.int32) * K)[:, None]
    g = _sc_gather(tab.reshape(B * K, 128), idx.reshape(NRG // 128, 128))
    g = g.reshape(B, P, 128)

    w = _wprep(e2)
    st = _taps(g, w)
    out = _final(w, st.transpose(0, 2, 1))
    return out.reshape(B, C, H_OUT, W_OUT)


# speedup vs baseline: 323.6540x; 1.0030x over previous
import math

import functools

import jax
import jax.numpy as jnp
from jax import lax
from jax.experimental import pallas as pl
from jax.experimental.pallas import tpu as pltpu
from jax.experimental.pallas import tpu_sc as plsc

CONTRAST = 3.38589
SQRT2 = math.sqrt(2.0)
B, C, H_IN, W_IN = 2, 3, 384, 384
H_OUT, W_OUT = 384, 384
P = H_OUT * W_OUT
WR = 3
H_PAD = H_IN + 2 * WR + 1
W_PAD = W_IN + 2 * WR + 1
K = H_PAD * W_PAD

_SIG1 = math.tanh(0.5 * CONTRAST * 0.5)
_A3 = -3.0
_A2 = (45739.0 + 7164.0 * SQRT2) / 10319.0
_A0 = (-8926.0 - 14328.0 * SQRT2) / 10319.0
_MIR = (-103.0 - 36.0 * SQRT2) / (7.0 + 72.0 * SQRT2)

PT = 1024


def _inverse_sigmoid(q):
    sig0 = -_SIG1
    slope = (1.0 / _SIG1 + sig0) * 0.25 * CONTRAST
    oos = 1.0 / slope
    res_low = q * oos
    res_high = q * oos + (1.0 - oos)
    ssq = jnp.clip(2.0 * _SIG1 * q + sig0, -0.999999, 0.999999)
    res_mid = (2.0 / CONTRAST) * (0.5 * jnp.log((1.0 + ssq) / (1.0 - ssq))) + 0.5
    return jnp.where(q <= 0.0, res_low, jnp.where(q >= 1.0, res_high, res_mid))


def _extended_sigmoid(q):
    slope = (1.0 / _SIG1 - _SIG1) * 0.25 * CONTRAST
    res_low = slope * q
    res_high = slope * q + (1.0 - slope)
    res_mid = 0.5 / _SIG1 * jnp.tanh(0.5 * CONTRAST * q - 0.25 * CONTRAST) + 0.5
    return jnp.where(q <= 0.0, res_low, jnp.where(q >= 1.0, res_high, res_mid))


def _mitchell(x):
    ax = jnp.abs(x)
    ax2 = ax * ax
    ax3 = ax2 * ax
    v1 = (7.0 / 6.0) * ax3 - 2.0 * ax2 + 8.0 / 9.0
    v2 = (-7.0 / 18.0) * ax3 + 2.0 * ax2 - (10.0 / 3.0) * ax + 16.0 / 9.0
    m1 = (ax < 1.0).astype(x.dtype)
    m2 = ((ax >= 1.0) & (ax < 2.0)).astype(x.dtype)
    return v1 * m1 + v2 * m2


def _pack_body(x_ref, o_ref):
    x = x_ref[...]
    s = _inverse_sigmoid(x)
    ru = lax.bitcast_convert_type(x.astype(jnp.bfloat16), jnp.uint16)
    su = lax.bitcast_convert_type(s.astype(jnp.bfloat16), jnp.uint16)
    o_ref[...] = (ru.astype(jnp.uint32) << 16) | su.astype(jnp.uint32)


def _pack_planes(img_pad):
    return pl.pallas_call(
        _pack_body,
        grid=(B, C),
        in_specs=[pl.BlockSpec((1, 1, H_PAD, W_PAD), lambda b, c: (b, c, 0, 0))],
        out_specs=pl.BlockSpec((1, 1, H_PAD, W_PAD), lambda b, c: (b, c, 0, 0)),
        out_shape=jax.ShapeDtypeStruct((B, C, H_PAD, W_PAD), jnp.uint32),
    )(img_pad)


NRG = B * P
NW = 32
GPW = NRG // NW
IDXR_PW = GPW // 128
SUBG = 4
CH = SUBG * 128
NCHUNK = GPW // CH


def _gather_body(tab_hbm, idx_hbm, out_hbm, idx_v, rows_v, sem):
    wid = lax.axis_index("s") * 2 + lax.axis_index("c")
    pltpu.sync_copy(idx_hbm.at[pl.ds(wid * IDXR_PW, IDXR_PW)], idx_v)

    def chunk(ci, carry):
        cops = [
            pltpu.async_copy(tab_hbm.at[idx_v.at[ci * SUBG + i]],
                             rows_v.at[pl.ds(i * 128, 128)], sem)
            for i in range(SUBG)
        ]
        for cp in cops:
            cp.wait()
        pltpu.sync_copy(
            rows_v, out_hbm.at[pl.ds(wid * GPW + ci * CH, CH)])
        return carry

    lax.fori_loop(0, NCHUNK, chunk, 0)


@functools.lru_cache(maxsize=None)
def _make_sc_gather():
    return pl.kernel(
        _gather_body,
        out_type=jax.ShapeDtypeStruct((NRG, 128), jnp.uint32),
        mesh=plsc.VectorSubcoreMesh(core_axis_name="c", subcore_axis_name="s"),
        scratch_types=[
            pltpu.VMEM((IDXR_PW, 128), jnp.int32),
            pltpu.VMEM((CH, 128), jnp.uint32),
            pltpu.SemaphoreType.DMA,
        ],
    )


def _sc_gather(tab, idx):
    return _make_sc_gather()(tab, idx)


KP = 153088


def _transp_body(x_ref, o_ref):
    o_ref[0] = jnp.transpose(x_ref[0])


def _transp(x):
    return pl.pallas_call(
        _transp_body,
        grid=(B, KP // 512),
        in_specs=[pl.BlockSpec((1, 128, 512), lambda b, i: (b, 0, i))],
        out_specs=pl.BlockSpec((1, 512, 128), lambda b, i: (b, i, 0)),
        out_shape=jax.ShapeDtypeStruct((B, KP, 128), jnp.uint32),
    )(x)


PTA = 2048


def _wprep_body(e_ref, o_ref):
    f32 = jnp.float32
    e = e_ref[0]
    j00 = e[0:1]
    j01 = e[1:2]
    j10 = e[2:3]
    j11 = e[3:4]
    fx = e[4:5]
    fy = e[5:6]

    det = j00 * j11 - j01 * j10 + 1e-8
    a = j11 / det
    b = -j01 / det
    c = -j10 / det
    d = j00 / det
    n11 = a * a + b * b
    n12 = a * c + b * d
    n22 = c * c + d * d
    frob = n11 + n22
    disc = frob * frob - 4.0 / (det * det)
    sq = jnp.sqrt(jnp.maximum(disc, 0.0))
    s1 = 0.5 * (frob + sq)
    major = jnp.sqrt(jnp.maximum(s1, 1.0))
    minor = jnp.sqrt(jnp.maximum(0.5 * (frob - sq), 1.0))
    d1 = s1 - n11
    d2 = s1 - n22
    cond = d1 * d1 >= d2 * d2
    t11 = jnp.where(cond, n12, d2)
    t21 = jnp.where(cond, d1, n12)
    norm = jnp.sqrt(t11 * t11 + t21 * t21)
    pos = norm > 0.0
    sn = jnp.where(pos, norm, 1.0)
    u11 = jnp.where(pos, t11 / sn, 1.0)
    u21 = jnp.where(pos, t21 / sn, 0.0)
    cmx = u11 / major
    cmy = u21 / major
    cnx = -u21 / minor
    cny = u11 / minor
    theta = 1.0 / (major * minor)
    need = ((frob + sq) > 2.0).astype(f32)

    mx = [_mitchell(fx - (xs - 2.0)) for xs in range(7)]
    my = [_mitchell(fy - (dy - 2.0)) for dy in range(6)]
    zero = jnp.zeros_like(fx)
    o_ref[0] = jnp.concatenate(
        mx + my + [cmx, cmy, cnx, cny, fx, fy, theta, need,
                   zero, zero, zero], axis=0)


def _wprep(e2):
    return pl.pallas_call(
        _wprep_body,
        grid=(B, P // PTA),
        in_specs=[pl.BlockSpec((1, 6, PTA), lambda b, i: (b, 0, i))],
        out_specs=pl.BlockSpec((1, 24, PTA), lambda b, i: (b, 0, i)),
        out_shape=jax.ShapeDtypeStruct((B, 24, P), jnp.float32),
    )(e2)


def _taps_body(g_ref, w_ref, o_ref):
    f32 = jnp.float32
    lane = lax.broadcasted_iota(jnp.int32, (1, 128), 1)
    xsl = (lane % 21) // 3
    oxl = (xsl - 2).astype(f32)
    oyl = ((lane // 21) - 2).astype(f32)

    g = g_ref[0]
    raw = lax.bitcast_convert_type(g & jnp.uint32(0xFFFF0000), f32)
    sig = lax.bitcast_convert_type(g << 16, f32)

    wt = jnp.transpose(w_ref[0])
    xtake = xsl
    ytake = jnp.minimum(lane // 21, 5) + 7
    mx_e = jnp.take_along_axis(wt, jnp.broadcast_to(xtake, wt.shape[:1] + (128,)), axis=1)
    my_e = jnp.take_along_axis(wt, jnp.broadcast_to(ytake, wt.shape[:1] + (128,)), axis=1)
    wm = mx_e * my_e
    cmx = wt[:, 13:14]
    cmy = wt[:, 14:15]
    cnx = wt[:, 15:16]
    cny = wt[:, 16:17]
    fx = wt[:, 17:18]
    fy = wt[:, 18:19]

    rel_x = fx - oxl
    rel_y = fy - oyl
    q1 = rel_x * cmx + rel_y * cmy
    q2 = rel_x * cnx + rel_y * cny
    r2 = q1 * q1 + q2 * q2
    r = jnp.sqrt(r2 + 1e-8)
    w_in = r2 * (_A3 * r + _A2) + _A0
    w_out = (r + _MIR) * (r - 2.0) * (r - 2.0)
    we = (w_in * (r2 < 1.0).astype(f32)
          + w_out * ((r2 >= 1.0) & (r2 < 4.0)).astype(f32))

    pm = sig * wm
    pe = raw * we

    ri = lax.broadcasted_iota(jnp.int32, (128, 8), 0)
    ci = lax.broadcasted_iota(jnp.int32, (128, 8), 1)
    ch_r = ri % 3
    xs_r = (ri % 21) // 3
    val_r = (ri < 126) & (xs_r < 6)
    m1 = ((ci < 3) & (ch_r == ci) & val_r).astype(f32)
    m2 = ((ci >= 3) & (ci < 6) & (ch_r == ci - 3) & val_r).astype(f32)
    m3 = ((ci == 6) & (ch_r == 0) & val_r).astype(f32)
    out = (jnp.dot(pm, m1, preferred_element_type=f32)
           + jnp.dot(pe, m2, preferred_element_type=f32)
           + jnp.dot(we, m3, preferred_element_type=f32))
    o_ref[0] = out


def _taps(g, w):
    return pl.pallas_call(
        _taps_body,
        grid=(B, P // PT),
        in_specs=[
            pl.BlockSpec((1, PT, 128), lambda b, i: (b, i, 0)),
            pl.BlockSpec((1, 24, PT), lambda b, i: (b, 0, i)),
        ],
        out_specs=pl.BlockSpec((1, PT, 8), lambda b, i: (b, i, 0)),
        out_shape=jax.ShapeDtypeStruct((B, P, 8), jnp.float32),
    )(g, w)


def _final_body(w_ref, s_ref, o_ref):
    theta = w_ref[0, 3:4]
    need = w_ref[0, 4:5]
    msum = s_ref[0, 0:3]
    esum = s_ref[0, 3:6]
    wsum = s_ref[0, 6:7] + 1e-8
    mv = _extended_sigmoid(msum)
    ev = esum / wsum
    bl = theta * mv + (1.0 - theta) * ev
    o_ref[0] = jnp.where(need > 0.5, bl, mv)


def _final(w, st):
    return pl.pallas_call(
        _final_body,
        grid=(B, P // PTA),
        in_specs=[
            pl.BlockSpec((1, 8, PTA), lambda b, i: (b, 2, i)),
            pl.BlockSpec((1, 8, PTA), lambda b, i: (b, 0, i)),
        ],
        out_specs=pl.BlockSpec((1, 3, PTA), lambda b, i: (b, 0, i)),
        out_shape=jax.ShapeDtypeStruct((B, 3, P), jnp.float32),
    )(w, st)


def kernel(image, grid):
    f32 = jnp.float32
    img_pad = jnp.pad(image, ((0, 0), (0, 0), (WR, WR + 1), (WR, WR + 1)),
                      mode="edge")
    vpk = _pack_planes(img_pad)

    flat = vpk.reshape(B, C, K)
    flatp = jnp.pad(flat, ((0, 0), (0, 0), (0, KP - K + 6 * W_PAD + 7)))
    tabt = jnp.stack([flatp[:, :, dy * W_PAD + xs: dy * W_PAD + xs + KP]
                      for dy in range(6) for xs in range(7)], axis=1)
    tabt = tabt.reshape(B, 126, KP)
    tabt = jnp.concatenate([tabt, jnp.zeros((B, 2, KP), jnp.uint32)], axis=1)
    tab = _transp(tabt)

    gx = grid[..., 0].reshape(B, P)
    gy = grid[..., 1].reshape(B, P)
    ix = jnp.clip(jnp.floor(gx).astype(jnp.int32), 0, W_IN - 2)
    iy = jnp.clip(jnp.floor(gy).astype(jnp.int32), 0, H_IN - 2)
    fx = gx - (ix.astype(f32) + 0.5)
    fy = gy - (iy.astype(f32) + 0.5)
    base = (iy + 1) * W_PAD + (ix + 1)

    gpx = jnp.pad(grid, ((0, 0), (0, 0), (1, 1), (0, 0)), mode="edge")
    dx = (gpx[:, :, 2:, :] - gpx[:, :, :-2, :]) * 0.5
    gpy = jnp.pad(grid, ((0, 0), (1, 1), (0, 0), (0, 0)), mode="edge")
    dy = (gpy[:, 2:, :, :] - gpy[:, :-2, :, :]) * 0.5
    e2 = jnp.stack([
        dx[..., 0].reshape(B, P), dy[..., 0].reshape(B, P),
        dx[..., 1].reshape(B, P), dy[..., 1].reshape(B, P),
        fx, fy,
    ], axis=1)

    idx = base + (jnp.arange(B, dtype=jnp.int32) * KP)[:, None]
    g = _sc_gather(tab.reshape(B * KP, 128), idx.reshape(NRG // 128, 128))
    g = g.reshape(B, P, 128)

    w = _wprep(e2)
    st = _taps(g, w)
    out = _final(w, st.transpose(0, 2, 1))
    return out.reshape(B, C, H_OUT, W_OUT)
